# baseline (speedup 1.0000x reference)
import functools

import jax
import jax.numpy as jnp
from jax.experimental import pallas as pl
from jax.experimental.pallas import tpu as pltpu

_IN = 512
_NN = 32768
_TS = 2048
_MM = 3
_NB = 16
_B = 1024

_WT = 1024


def _pow_terms(n):
    t2 = n ** 2.0
    t4 = n ** 4.0
    t8 = n ** 8.0
    return t2, t4, t8


def _bucket_from_bits(proj, out_rows_as_lanes):
    bits = (proj > 0).astype(jnp.float32)
    io = jax.lax.broadcasted_iota(jnp.int32, (1, _NB), 1)
    pw = jnp.where(io < 11, jax.lax.shift_left(jnp.int32(1), io), 0).astype(
        jnp.float32
    )
    if out_rows_as_lanes:
        return jax.lax.dot_general(
            pw, bits, (((1,), (1,)), ((), ())),
            preferred_element_type=jnp.float32)
    return jnp.sum(bits * pw, axis=1, keepdims=True)


def _qhash_kernel(x_ref, a_ref, qb_ref):
    x = x_ref[...]
    n = jnp.sqrt(jnp.sum(x * x, axis=1, keepdims=True))
    xn = x / n
    aug = jnp.concatenate(
        [xn, jnp.full((x.shape[0], _MM), 0.5, jnp.float32)], axis=1)
    proj = jax.lax.dot_general(
        aug, a_ref[...], (((1,), (1,)), ((), ())),
        preferred_element_type=jnp.float32)
    qb_ref[...] = _bucket_from_bits(proj, out_rows_as_lanes=False)


def _main_kernel(x_ref, xb_ref, w_ref, a_ref, qb_ref, out_ref):
    w = w_ref[...]
    ones = jnp.ones((1, _IN), jnp.float32)
    n2 = jax.lax.dot_general(
        w * w, ones, (((1,), (1,)), ((), ())),
        preferred_element_type=jnp.float32)
    n = jnp.sqrt(n2)
    t2, t4, t8 = _pow_terms(n)
    aug = jnp.concatenate([w, t2, t4, t8], axis=1)
    proj = jax.lax.dot_general(
        aug, a_ref[...], (((1,), (1,)), ((), ())),
        preferred_element_type=jnp.float32)
    wb = _bucket_from_bits(proj, out_rows_as_lanes=True)
    scores = jax.lax.dot_general(
        xb_ref[...], w.astype(jnp.bfloat16), (((1,), (1,)), ((), ())),
        preferred_element_type=jnp.float32)
    mask = qb_ref[...] == wb
    out_ref[...] = jnp.where(mask, scores, 0.0)


def kernel(x, weight, A, mode):
    del mode
    qb = pl.pallas_call(
        _qhash_kernel,
        out_shape=jax.ShapeDtypeStruct((_B, 1), jnp.float32),
    )(x, A)
    grid = _NN // _WT
    xb = x.astype(jnp.bfloat16)
    out = pl.pallas_call(
        _main_kernel,
        grid=(grid,),
        in_specs=[
            pl.BlockSpec((_B, _IN), lambda i: (0, 0)),
            pl.BlockSpec((_B, _IN), lambda i: (0, 0)),
            pl.BlockSpec((_WT, _IN), lambda i: (i, 0)),
            pl.BlockSpec((_NB, _IN + _MM), lambda i: (0, 0)),
            pl.BlockSpec((_B, 1), lambda i: (0, 0)),
        ],
        out_specs=pl.BlockSpec((_B, _WT), lambda i: (0, i)),
        out_shape=jax.ShapeDtypeStruct((_B, _NN), jnp.float32),
    )(x, xb, weight, A, qb)
    return out


# speedup vs baseline: 1.1274x; 1.1274x over previous
import functools

import jax
import jax.numpy as jnp
from jax.experimental import pallas as pl
from jax.experimental.pallas import tpu as pltpu

_IN = 512
_NN = 32768
_TS = 2048
_MM = 3
_NB = 16
_B = 1024

_WT = 2048


def _pow_terms(n):
    t2 = n ** 2.0
    t4 = n ** 4.0
    t8 = n ** 8.0
    return t2, t4, t8


def _bucket_from_bits(proj, out_rows_as_lanes):
    bits = (proj > 0).astype(jnp.float32)
    io = jax.lax.broadcasted_iota(jnp.int32, (1, _NB), 1)
    pw = jnp.where(io < 11, jax.lax.shift_left(jnp.int32(1), io), 0).astype(
        jnp.float32
    )
    if out_rows_as_lanes:
        return jax.lax.dot_general(
            pw, bits, (((1,), (1,)), ((), ())),
            preferred_element_type=jnp.float32)
    return jnp.sum(bits * pw, axis=1, keepdims=True)


def _qhash_kernel(x_ref, a_ref, qb_ref):
    x = x_ref[...]
    n = jnp.sqrt(jnp.sum(x * x, axis=1, keepdims=True))
    xn = x / n
    aug = jnp.concatenate(
        [xn, jnp.full((x.shape[0], _MM), 0.5, jnp.float32)], axis=1)
    proj = jax.lax.dot_general(
        aug, a_ref[...], (((1,), (1,)), ((), ())),
        preferred_element_type=jnp.float32)
    qb_ref[...] = _bucket_from_bits(proj, out_rows_as_lanes=False)


def _main_kernel(x_ref, xb_ref, w_ref, a_ref, qb_ref, out_ref):
    w = w_ref[...]
    n = jnp.sqrt(jnp.sum(w * w, axis=1, keepdims=True))
    t2, t4, t8 = _pow_terms(n)
    aug = jnp.concatenate([w, t2, t4, t8], axis=1)
    proj = jax.lax.dot_general(
        aug, a_ref[...], (((1,), (1,)), ((), ())),
        preferred_element_type=jnp.float32)
    wb = _bucket_from_bits(proj, out_rows_as_lanes=True)
    scores = jax.lax.dot_general(
        xb_ref[...], w.astype(jnp.bfloat16), (((1,), (1,)), ((), ())),
        preferred_element_type=jnp.float32)
    mask = qb_ref[...] == wb
    out_ref[...] = jnp.where(mask, scores, 0.0)


def kernel(x, weight, A, mode):
    del mode
    qb = pl.pallas_call(
        _qhash_kernel,
        out_shape=jax.ShapeDtypeStruct((_B, 1), jnp.float32),
    )(x, A)
    grid = _NN // _WT
    xb = x.astype(jnp.bfloat16)
    out = pl.pallas_call(
        _main_kernel,
        grid=(grid,),
        in_specs=[
            pl.BlockSpec((_B, _IN), lambda i: (0, 0)),
            pl.BlockSpec((_B, _IN), lambda i: (0, 0)),
            pl.BlockSpec((_WT, _IN), lambda i: (i, 0)),
            pl.BlockSpec((_NB, _IN + _MM), lambda i: (0, 0)),
            pl.BlockSpec((_B, 1), lambda i: (0, 0)),
        ],
        out_specs=pl.BlockSpec((_B, _WT), lambda i: (0, i)),
        out_shape=jax.ShapeDtypeStruct((_B, _NN), jnp.float32),
    )(x, xb, weight, A, qb)
    return out
